# Initial kernel scaffold; baseline (speedup 1.0000x reference)
import jax
import jax.numpy as jnp
from jax.experimental import pallas as pl


def kernel(x, edge_index, W, att_src, att_dst, bias, gamma, beta):
    raise NotImplementedError("write your pallas kernel here")


# speedup vs baseline: 23.5855x; 23.5855x over previous
import functools

import jax
import jax.numpy as jnp
from jax import lax
from jax.experimental import pallas as pl
from jax.experimental.pallas import tpu as pltpu
from jax.experimental.pallas import tpu_sc as plsc

N = 10000
D = 128
E = 320000

NC = 2
NS = 16
NW = NC * NS
EW = E // NW
BLK = 80
NBLK = EW // BLK
NPAD = 10240
NPT = NPAD // NS
DPT = NPAD // NS

_NEG_SLOPE = 0.2


def _proj_body(x_ref, w_ref, asrc_ref, adst_ref,
               h_ref, as_ref, ad_ref, m_ref, ms_ref, md_ref):
    i = pl.program_id(0)
    h = jnp.dot(x_ref[...], w_ref[...], preferred_element_type=jnp.float32)
    h_ref[...] = h
    a_s = jnp.dot(h, asrc_ref[...], preferred_element_type=jnp.float32)
    a_d = jnp.dot(h, adst_ref[...], preferred_element_type=jnp.float32)
    as_ref[...] = a_s
    ad_ref[...] = a_d
    bs = jnp.max(a_s)
    bd = jnp.max(a_d)

    @pl.when(i == 0)
    def _():
        ms_ref[0] = bs
        md_ref[0] = bd

    @pl.when(i > 0)
    def _():
        ms_ref[0] = jnp.maximum(ms_ref[0], bs)
        md_ref[0] = jnp.maximum(md_ref[0], bd)

    @pl.when(i == pl.num_programs(0) - 1)
    def _():
        t = ms_ref[0] + md_ref[0]
        m_ref[0, 0] = jnp.where(t > 0, t, _NEG_SLOPE * t)


def _project(x, w, asrc, adst):
    rows = 1000
    grid = (N // rows,)
    return pl.pallas_call(
        _proj_body,
        grid=grid,
        in_specs=[
            pl.BlockSpec((rows, D), lambda i: (i, 0)),
            pl.BlockSpec((D, D), lambda i: (0, 0)),
            pl.BlockSpec((D, 1), lambda i: (0, 0)),
            pl.BlockSpec((D, 1), lambda i: (0, 0)),
        ],
        out_specs=[
            pl.BlockSpec((rows, D), lambda i: (i, 0)),
            pl.BlockSpec((rows, 1), lambda i: (i, 0)),
            pl.BlockSpec((rows, 1), lambda i: (i, 0)),
            pl.BlockSpec(memory_space=pltpu.SMEM),
        ],
        out_shape=[
            jax.ShapeDtypeStruct((N, D), jnp.float32),
            jax.ShapeDtypeStruct((N, 1), jnp.float32),
            jax.ShapeDtypeStruct((N, 1), jnp.float32),
            jax.ShapeDtypeStruct((1, 1), jnp.float32),
        ],
        scratch_shapes=[
            pltpu.SMEM((1,), jnp.float32),
            pltpu.SMEM((1,), jnp.float32),
        ],
    )(x, w, asrc, adst)


def _edge_body(h_hbm, src_hbm, dst_hbm, as_hbm, ad_hbm, m_hbm,
               out_hbm, den_hbm,
               asv, adv, mv, sidx, didx, wv, rows,
               acc_s, den_s, sem):
    c = lax.axis_index("c")
    s = lax.axis_index("s")
    wid = s * NC + c
    z16f = jnp.zeros((16,), jnp.float32)

    pltpu.sync_copy(as_hbm, asv)
    pltpu.sync_copy(ad_hbm, adv)
    pltpu.sync_copy(m_hbm, mv)
    mvec = mv[...]

    def zero_rows(i, carry):
        for dch in range(8):
            rows[i, pl.ds(dch * 16, 16)] = z16f
        return carry
    lax.fori_loop(0, BLK, zero_rows, 0)
    for q in range(BLK // 16):
        wv[pl.ds(16 * q, 16)] = z16f
    for off in range(0, NPT, BLK):
        pltpu.sync_copy(rows, acc_s.at[pl.ds(NPT * s + off, BLK)])
        pltpu.sync_copy(wv, den_s.at[pl.ds(DPT * s + off, BLK)])
    plsc.subcore_barrier()

    base0 = wid * EW

    def block_body(blk, carry):
        base = base0 + blk * BLK
        pltpu.sync_copy(src_hbm.at[pl.ds(base, BLK)], sidx)
        pltpu.sync_copy(dst_hbm.at[pl.ds(base, BLK)], didx)

        cp = pltpu.async_copy(h_hbm.at[sidx], rows, sem)

        for q in range(BLK // 16):
            s_ids = sidx[pl.ds(16 * q, 16)]
            d_ids = didx[pl.ds(16 * q, 16)]
            av = plsc.load_gather(asv, [s_ids])
            bv = plsc.load_gather(adv, [d_ids])
            e = av + bv
            e = jnp.where(e > 0, e, _NEG_SLOPE * e)
            wv[pl.ds(16 * q, 16)] = jnp.exp(e - mvec)

        cp.wait()

        def mul_body(ei, carry2):
            wb = plsc.load_gather(wv, [jnp.full((16,), ei, jnp.int32)])
            for dch in range(8):
                sl = pl.ds(dch * 16, 16)
                rows[ei, sl] = rows[ei, sl] * wb
            return carry2
        lax.fori_loop(0, BLK, mul_body, 0)

        pltpu.sync_copy(rows, acc_s.at[didx], add=True)
        pltpu.sync_copy(wv, den_s.at[didx], add=True)
        return carry

    lax.fori_loop(0, NBLK, block_body, 0)

    plsc.subcore_barrier()
    pltpu.sync_copy(acc_s.at[pl.ds(NPT * s, NPT)],
                    out_hbm.at[c, pl.ds(NPT * s, NPT)])
    pltpu.sync_copy(den_s.at[pl.ds(DPT * s, DPT)],
                    den_hbm.at[c, pl.ds(DPT * s, DPT)])


def _edge_scatter(h, src, dst, a_s, a_d, m16):
    mesh = plsc.VectorSubcoreMesh(core_axis_name="c", subcore_axis_name="s",
                                  num_cores=NC, num_subcores=NS)
    kern = pl.kernel(
        _edge_body,
        out_type=(
            jax.ShapeDtypeStruct((NC, NPAD, D), jnp.float32),
            jax.ShapeDtypeStruct((NC, NPAD), jnp.float32),
        ),
        mesh=mesh,
        compiler_params=pltpu.CompilerParams(needs_layout_passes=False),
        scratch_types=[
            pltpu.VMEM((N,), jnp.float32),
            pltpu.VMEM((N,), jnp.float32),
            pltpu.VMEM((16,), jnp.float32),
            pltpu.VMEM((BLK,), jnp.int32),
            pltpu.VMEM((BLK,), jnp.int32),
            pltpu.VMEM((BLK,), jnp.float32),
            pltpu.VMEM((BLK, D), jnp.float32),
            pltpu.VMEM_SHARED((NPAD, D), jnp.float32),
            pltpu.VMEM_SHARED((NPAD,), jnp.float32),
            pltpu.SemaphoreType.DMA,
        ],
    )
    return kern(h, src, dst, a_s, a_d, m16)


def _final_body(p_ref, d_ref, h_ref, as_ref, ad_ref, m_ref,
                bias_ref, gamma_ref, beta_ref, y_ref):
    mval = m_ref[0, 0]
    a = as_ref[...] + ad_ref[...]
    e = jnp.where(a > 0, a, _NEG_SLOPE * a)
    wl = jnp.exp(e - mval)
    raw = p_ref[0] + p_ref[1] + wl * h_ref[...]
    den = d_ref[0] + d_ref[1] + wl
    z = raw / (den + 1e-16) + bias_ref[...]
    mean = jnp.mean(z, axis=0, keepdims=True)
    var = jnp.mean((z - mean) ** 2, axis=0, keepdims=True)
    y = (z - mean) * lax.rsqrt(var + 1e-5) * gamma_ref[...] + beta_ref[...]
    y_ref[...] = jnp.maximum(y, 0.0)


def _finalize(partials, den2, h, a_s, a_d, m, bias, gamma, beta):
    return pl.pallas_call(
        _final_body,
        in_specs=[
            pl.BlockSpec((NC, N, D), lambda: (0, 0, 0)),
            pl.BlockSpec((NC, N, 1), lambda: (0, 0, 0)),
            pl.BlockSpec((N, D), lambda: (0, 0)),
            pl.BlockSpec((N, 1), lambda: (0, 0)),
            pl.BlockSpec((N, 1), lambda: (0, 0)),
            pl.BlockSpec(memory_space=pltpu.SMEM),
            pl.BlockSpec((1, D), lambda: (0, 0)),
            pl.BlockSpec((1, D), lambda: (0, 0)),
            pl.BlockSpec((1, D), lambda: (0, 0)),
        ],
        out_specs=pl.BlockSpec((N, D), lambda: (0, 0)),
        out_shape=jax.ShapeDtypeStruct((N, D), jnp.float32),
    )(partials, den2, h, a_s, a_d, m, bias, gamma, beta)


def kernel(x, edge_index, W, att_src, att_dst, bias, gamma, beta):
    h, a_s, a_d, m = _project(x, W,
                              att_src.reshape(D, 1), att_dst.reshape(D, 1))
    m16 = jnp.broadcast_to(m.reshape(()), (16,))
    src = edge_index[0]
    dst = edge_index[1]
    partials, denp = _edge_scatter(h, src, dst,
                                   a_s.reshape(N), a_d.reshape(N), m16)
    partials = partials[:, :N]
    den2 = denp[:, :N].reshape(NC, N, 1)
    return _finalize(partials, den2, h, a_s, a_d, m,
                     bias.reshape(1, D), gamma.reshape(1, D),
                     beta.reshape(1, D))


# speedup vs baseline: 30.8306x; 1.3072x over previous
import functools

import jax
import jax.numpy as jnp
from jax import lax
from jax.experimental import pallas as pl
from jax.experimental.pallas import tpu as pltpu
from jax.experimental.pallas import tpu_sc as plsc

N = 10000
D = 128
E = 320000

NC = 2
NS = 16
NW = NC * NS
EW = E // NW
BLK = 80
NBLK = EW // BLK
NPAD = 10240
NPT = NPAD // NS
DPT = NPAD // NS

_NEG_SLOPE = 0.2


def _proj_body(x_ref, w_ref, asrc_ref, adst_ref,
               h_ref, as_ref, ad_ref, m_ref, ms_ref, md_ref):
    i = pl.program_id(0)
    h = jnp.dot(x_ref[...], w_ref[...], preferred_element_type=jnp.float32)
    h_ref[...] = h
    a_s = jnp.dot(h, asrc_ref[...], preferred_element_type=jnp.float32)
    a_d = jnp.dot(h, adst_ref[...], preferred_element_type=jnp.float32)
    as_ref[...] = a_s
    ad_ref[...] = a_d
    bs = jnp.max(a_s)
    bd = jnp.max(a_d)

    @pl.when(i == 0)
    def _():
        ms_ref[0] = bs
        md_ref[0] = bd

    @pl.when(i > 0)
    def _():
        ms_ref[0] = jnp.maximum(ms_ref[0], bs)
        md_ref[0] = jnp.maximum(md_ref[0], bd)

    @pl.when(i == pl.num_programs(0) - 1)
    def _():
        t = ms_ref[0] + md_ref[0]
        m_ref[0, 0] = jnp.where(t > 0, t, _NEG_SLOPE * t)


def _project(x, w, asrc, adst):
    rows = 1000
    grid = (N // rows,)
    return pl.pallas_call(
        _proj_body,
        grid=grid,
        in_specs=[
            pl.BlockSpec((rows, D), lambda i: (i, 0)),
            pl.BlockSpec((D, D), lambda i: (0, 0)),
            pl.BlockSpec((D, 1), lambda i: (0, 0)),
            pl.BlockSpec((D, 1), lambda i: (0, 0)),
        ],
        out_specs=[
            pl.BlockSpec((rows, D), lambda i: (i, 0)),
            pl.BlockSpec((rows, 1), lambda i: (i, 0)),
            pl.BlockSpec((rows, 1), lambda i: (i, 0)),
            pl.BlockSpec(memory_space=pltpu.SMEM),
        ],
        out_shape=[
            jax.ShapeDtypeStruct((N, D), jnp.float32),
            jax.ShapeDtypeStruct((N, 1), jnp.float32),
            jax.ShapeDtypeStruct((N, 1), jnp.float32),
            jax.ShapeDtypeStruct((1, 1), jnp.float32),
        ],
        scratch_shapes=[
            pltpu.SMEM((1,), jnp.float32),
            pltpu.SMEM((1,), jnp.float32),
        ],
    )(x, w, asrc, adst)


def _edge_body(h_hbm, src_hbm, dst_hbm, as_hbm, ad_hbm, m_hbm,
               out_hbm, den_hbm,
               asv, adv, mv, sidx, didx, wv, rows,
               acc_s, den_s, semi, semg, sems0, sems1):
    c = lax.axis_index("c")
    s = lax.axis_index("s")
    wid = s * NC + c
    z16f = jnp.zeros((16,), jnp.float32)
    sems = (sems0, sems1)

    pltpu.sync_copy(as_hbm, asv)
    pltpu.sync_copy(ad_hbm, adv)
    pltpu.sync_copy(m_hbm, mv)
    mvec = mv[...]

    def zero_rows(i, carry):
        for dch in range(8):
            rows[0, i, pl.ds(dch * 16, 16)] = z16f
        return carry
    lax.fori_loop(0, BLK, zero_rows, 0)
    for q in range(BLK // 16):
        wv[0, pl.ds(16 * q, 16)] = z16f
    for off in range(0, NPT, BLK):
        pltpu.sync_copy(rows.at[0], acc_s.at[pl.ds(NPT * s + off, BLK)])
        pltpu.sync_copy(wv.at[0], den_s.at[pl.ds(DPT * s + off, BLK)])
    plsc.subcore_barrier()

    base0 = wid * EW

    def issue_idx(blk, b):
        base = base0 + blk * BLK
        pltpu.async_copy(src_hbm.at[pl.ds(base, BLK)], sidx.at[b], semi)
        pltpu.async_copy(dst_hbm.at[pl.ds(base, BLK)], didx.at[b], semi)

    def wait_idx(blk, b):
        base = base0 + blk * BLK
        pltpu.make_async_copy(src_hbm.at[pl.ds(base, BLK)], sidx.at[b],
                              semi).wait()
        pltpu.make_async_copy(dst_hbm.at[pl.ds(base, BLK)], didx.at[b],
                              semi).wait()

    def wait_scat(b):
        pltpu.make_async_copy(rows.at[b], acc_s.at[didx.at[b]],
                              sems[b]).wait()
        pltpu.make_async_copy(wv.at[b], den_s.at[didx.at[b]],
                              sems[b]).wait()

    def do_block(blk, b, first=False, prefetch=True):
        wait_idx(blk, b)
        cp = pltpu.async_copy(h_hbm.at[sidx.at[b]], rows.at[b], semg)

        for q in range(BLK // 16):
            s_ids = sidx[b, pl.ds(16 * q, 16)]
            d_ids = didx[b, pl.ds(16 * q, 16)]
            av = plsc.load_gather(asv, [s_ids])
            bv = plsc.load_gather(adv, [d_ids])
            e = av + bv
            e = jnp.where(e > 0, e, _NEG_SLOPE * e)
            wv[b, pl.ds(16 * q, 16)] = jnp.exp(e - mvec)

        cp.wait()

        bvec = jnp.full((16,), b, jnp.int32)

        def mul_body(ei, carry2):
            for u in range(2):
                eu = 2 * ei + u
                wb = plsc.load_gather(wv, [bvec, jnp.full((16,), eu,
                                                          jnp.int32)])
                for dch in range(8):
                    sl = pl.ds(dch * 16, 16)
                    rows[b, eu, sl] = rows[b, eu, sl] * wb
            return carry2
        lax.fori_loop(0, BLK // 2, mul_body, 0)

        if not first:
            wait_scat(1 - b)
        if prefetch:
            issue_idx(blk + 1, 1 - b)

        pltpu.async_copy(rows.at[b], acc_s.at[didx.at[b]], sems[b], add=True)
        pltpu.async_copy(wv.at[b], den_s.at[didx.at[b]], sems[b], add=True)

    issue_idx(0, 0)
    do_block(0, 0, first=True)
    do_block(1, 1)

    def loop_body(g, carry):
        do_block(2 * g, 0)
        do_block(2 * g + 1, 1)
        return carry

    lax.fori_loop(1, NBLK // 2, loop_body, 0)
    do_block(NBLK - 1, 0, prefetch=False)
    wait_scat(0)

    plsc.subcore_barrier()
    pltpu.sync_copy(acc_s.at[pl.ds(NPT * s, NPT)],
                    out_hbm.at[c, pl.ds(NPT * s, NPT)])
    pltpu.sync_copy(den_s.at[pl.ds(DPT * s, DPT)],
                    den_hbm.at[c, pl.ds(DPT * s, DPT)])


def _edge_scatter(h, src, dst, a_s, a_d, m16):
    mesh = plsc.VectorSubcoreMesh(core_axis_name="c", subcore_axis_name="s",
                                  num_cores=NC, num_subcores=NS)
    kern = pl.kernel(
        _edge_body,
        out_type=(
            jax.ShapeDtypeStruct((NC, NPAD, D), jnp.float32),
            jax.ShapeDtypeStruct((NC, NPAD), jnp.float32),
        ),
        mesh=mesh,
        compiler_params=pltpu.CompilerParams(needs_layout_passes=False),
        scratch_types=[
            pltpu.VMEM((N,), jnp.float32),
            pltpu.VMEM((N,), jnp.float32),
            pltpu.VMEM((16,), jnp.float32),
            pltpu.VMEM((2, BLK), jnp.int32),
            pltpu.VMEM((2, BLK), jnp.int32),
            pltpu.VMEM((2, BLK), jnp.float32),
            pltpu.VMEM((2, BLK, D), jnp.float32),
            pltpu.VMEM_SHARED((NPAD, D), jnp.float32),
            pltpu.VMEM_SHARED((NPAD,), jnp.float32),
            pltpu.SemaphoreType.DMA,
            pltpu.SemaphoreType.DMA,
            pltpu.SemaphoreType.DMA,
            pltpu.SemaphoreType.DMA,
        ],
    )
    return kern(h, src, dst, a_s, a_d, m16)


def _final_body(p_ref, d_ref, h_ref, as_ref, ad_ref, m_ref,
                bias_ref, gamma_ref, beta_ref, y_ref):
    mval = m_ref[0, 0]
    a = as_ref[...] + ad_ref[...]
    e = jnp.where(a > 0, a, _NEG_SLOPE * a)
    wl = jnp.exp(e - mval)
    raw = p_ref[0] + p_ref[1] + wl * h_ref[...]
    den = d_ref[0] + d_ref[1] + wl
    z = raw / (den + 1e-16) + bias_ref[...]
    mean = jnp.mean(z, axis=0, keepdims=True)
    var = jnp.mean((z - mean) ** 2, axis=0, keepdims=True)
    y = (z - mean) * lax.rsqrt(var + 1e-5) * gamma_ref[...] + beta_ref[...]
    y_ref[...] = jnp.maximum(y, 0.0)


def _finalize(partials, den2, h, a_s, a_d, m, bias, gamma, beta):
    return pl.pallas_call(
        _final_body,
        in_specs=[
            pl.BlockSpec((NC, N, D), lambda: (0, 0, 0)),
            pl.BlockSpec((NC, N, 1), lambda: (0, 0, 0)),
            pl.BlockSpec((N, D), lambda: (0, 0)),
            pl.BlockSpec((N, 1), lambda: (0, 0)),
            pl.BlockSpec((N, 1), lambda: (0, 0)),
            pl.BlockSpec(memory_space=pltpu.SMEM),
            pl.BlockSpec((1, D), lambda: (0, 0)),
            pl.BlockSpec((1, D), lambda: (0, 0)),
            pl.BlockSpec((1, D), lambda: (0, 0)),
        ],
        out_specs=pl.BlockSpec((N, D), lambda: (0, 0)),
        out_shape=jax.ShapeDtypeStruct((N, D), jnp.float32),
    )(partials, den2, h, a_s, a_d, m, bias, gamma, beta)


def kernel(x, edge_index, W, att_src, att_dst, bias, gamma, beta):
    h, a_s, a_d, m = _project(x, W,
                              att_src.reshape(D, 1), att_dst.reshape(D, 1))
    m16 = jnp.broadcast_to(m.reshape(()), (16,))
    src = edge_index[0]
    dst = edge_index[1]
    partials, denp = _edge_scatter(h, src, dst,
                                   a_s.reshape(N), a_d.reshape(N), m16)
    partials = partials[:, :N]
    den2 = denp[:, :N].reshape(NC, N, 1)
    return _finalize(partials, den2, h, a_s, a_d, m,
                     bias.reshape(1, D), gamma.reshape(1, D),
                     beta.reshape(1, D))


# speedup vs baseline: 45.3825x; 1.4720x over previous
import functools

import jax
import jax.numpy as jnp
from jax import lax
from jax.experimental import pallas as pl
from jax.experimental.pallas import tpu as pltpu
from jax.experimental.pallas import tpu_sc as plsc

N = 10000
D = 128
E = 320000

NC = 2
NS = 16
NW = NC * NS
EW = E // NW
BLK = 80
NBLK = EW // BLK
NPAD = 10240
NPT = NPAD // NS
DPT = NPAD // NS

_NEG_SLOPE = 0.2


def _proj_body(x_ref, w_ref, asrc_ref, adst_ref,
               h_ref, as_ref, ad_ref, m_ref, ms_ref, md_ref):
    i = pl.program_id(0)
    h = jnp.dot(x_ref[...], w_ref[...], preferred_element_type=jnp.float32)
    h_ref[...] = h
    a_s = jnp.dot(h, asrc_ref[...], preferred_element_type=jnp.float32)
    a_d = jnp.dot(h, adst_ref[...], preferred_element_type=jnp.float32)
    as_ref[...] = a_s
    ad_ref[...] = a_d
    bs = jnp.max(a_s)
    bd = jnp.max(a_d)

    @pl.when(i == 0)
    def _():
        ms_ref[0] = bs
        md_ref[0] = bd

    @pl.when(i > 0)
    def _():
        ms_ref[0] = jnp.maximum(ms_ref[0], bs)
        md_ref[0] = jnp.maximum(md_ref[0], bd)

    @pl.when(i == pl.num_programs(0) - 1)
    def _():
        t = ms_ref[0] + md_ref[0]
        m_ref[0, 0] = jnp.where(t > 0, t, _NEG_SLOPE * t)


def _project(x, w, asrc, adst):
    rows = 1000
    grid = (N // rows,)
    return pl.pallas_call(
        _proj_body,
        grid=grid,
        in_specs=[
            pl.BlockSpec((rows, D), lambda i: (i, 0)),
            pl.BlockSpec((D, D), lambda i: (0, 0)),
            pl.BlockSpec((D, 1), lambda i: (0, 0)),
            pl.BlockSpec((D, 1), lambda i: (0, 0)),
        ],
        out_specs=[
            pl.BlockSpec((rows, D), lambda i: (i, 0)),
            pl.BlockSpec((rows, 1), lambda i: (i, 0)),
            pl.BlockSpec((rows, 1), lambda i: (i, 0)),
            pl.BlockSpec(memory_space=pltpu.SMEM),
        ],
        out_shape=[
            jax.ShapeDtypeStruct((N, D), jnp.float32),
            jax.ShapeDtypeStruct((N, 1), jnp.float32),
            jax.ShapeDtypeStruct((N, 1), jnp.float32),
            jax.ShapeDtypeStruct((1, 1), jnp.float32),
        ],
        scratch_shapes=[
            pltpu.SMEM((1,), jnp.float32),
            pltpu.SMEM((1,), jnp.float32),
        ],
    )(x, w, asrc, adst)


def _edge_body(h_hbm, src_hbm, dst_hbm, as_hbm, ad_hbm, m_hbm,
               out_hbm, den_hbm,
               mv, sidx, didx, av_b, bv_b, wv, rows,
               acc_s, den_s, semi, sema, semg, sems):
    c = lax.axis_index("c")
    s = lax.axis_index("s")
    wid = s * NC + c
    z16f = jnp.zeros((16,), jnp.float32)

    pltpu.sync_copy(m_hbm, mv)
    mvec = mv[...]

    def zero_rows(i, carry):
        for dch in range(8):
            rows[0, i, pl.ds(dch * 16, 16)] = z16f
        return carry
    lax.fori_loop(0, BLK, zero_rows, 0)
    for q in range(BLK // 16):
        wv[0, pl.ds(16 * q, 16)] = z16f
    for off in range(0, NPT, BLK):
        pltpu.sync_copy(rows.at[0], acc_s.at[pl.ds(NPT * s + off, BLK)])
        pltpu.sync_copy(wv.at[0], den_s.at[pl.ds(DPT * s + off, BLK)])
    plsc.subcore_barrier()

    base0 = wid * EW

    def issue_idx(blk, p):
        base = base0 + blk * BLK
        pltpu.async_copy(src_hbm.at[pl.ds(base, BLK)], sidx.at[p], semi)
        pltpu.async_copy(dst_hbm.at[pl.ds(base, BLK)], didx.at[p], semi)

    def wait_idx(blk, p):
        base = base0 + blk * BLK
        pltpu.make_async_copy(src_hbm.at[pl.ds(base, BLK)], sidx.at[p],
                              semi).wait()
        pltpu.make_async_copy(dst_hbm.at[pl.ds(base, BLK)], didx.at[p],
                              semi).wait()

    def issue_gathers(p):
        pltpu.async_copy(h_hbm.at[sidx.at[p]], rows.at[p], semg)
        pltpu.async_copy(as_hbm.at[sidx.at[p]], av_b.at[p], sema)
        pltpu.async_copy(ad_hbm.at[didx.at[p]], bv_b.at[p], sema)

    def wait_agather(p):
        pltpu.make_async_copy(as_hbm.at[sidx.at[p]], av_b.at[p],
                              sema).wait()
        pltpu.make_async_copy(ad_hbm.at[didx.at[p]], bv_b.at[p],
                              sema).wait()

    def wait_rgather(p):
        pltpu.make_async_copy(h_hbm.at[sidx.at[p]], rows.at[p],
                              semg).wait()

    def wait_scat(p):
        pltpu.make_async_copy(rows.at[p], acc_s.at[didx.at[p]],
                              sems).wait()
        pltpu.make_async_copy(wv.at[p], den_s.at[didx.at[p]],
                              sems).wait()

    def do_block(blk, p, first=False, last=False, prefetch2=True):
        p1 = (p + 1) % 3
        p2 = (p + 2) % 3
        if not last:
            wait_idx(blk + 1, p1)
            issue_gathers(p1)

        wait_agather(p)
        for q in range(BLK // 16):
            e = av_b[p, pl.ds(16 * q, 16)] + bv_b[p, pl.ds(16 * q, 16)]
            e = jnp.where(e > 0, e, _NEG_SLOPE * e)
            wv[p, pl.ds(16 * q, 16)] = jnp.exp(e - mvec)

        wait_rgather(p)
        pvec = jnp.full((16,), p, jnp.int32)

        def mul_body(ei, carry2):
            for u in range(4):
                eu = 4 * ei + u
                wb = plsc.load_gather(wv, [pvec, jnp.full((16,), eu,
                                                          jnp.int32)])
                for dch in range(8):
                    sl = pl.ds(dch * 16, 16)
                    rows[p, eu, sl] = rows[p, eu, sl] * wb
            return carry2
        lax.fori_loop(0, BLK // 4, mul_body, 0)

        if not first:
            wait_scat(p2)
        if prefetch2 and not last:
            issue_idx(blk + 2, p2)

        pltpu.async_copy(rows.at[p], acc_s.at[didx.at[p]], sems, add=True)
        pltpu.async_copy(wv.at[p], den_s.at[didx.at[p]], sems, add=True)

    issue_idx(0, 0)
    wait_idx(0, 0)
    issue_gathers(0)
    issue_idx(1, 1)
    do_block(0, 0, first=True)

    def loop_body(g, carry):
        do_block(3 * g + 1, 1)
        do_block(3 * g + 2, 2)
        do_block(3 * g + 3, 0)
        return carry

    lax.fori_loop(0, (NBLK - 5) // 3, loop_body, 0)
    do_block(NBLK - 4, 1)
    do_block(NBLK - 3, 2)
    do_block(NBLK - 2, 0, prefetch2=False)
    do_block(NBLK - 1, 1, last=True)
    wait_scat(1)

    plsc.subcore_barrier()
    pltpu.sync_copy(acc_s.at[pl.ds(NPT * s, NPT)],
                    out_hbm.at[c, pl.ds(NPT * s, NPT)])
    pltpu.sync_copy(den_s.at[pl.ds(DPT * s, DPT)],
                    den_hbm.at[c, pl.ds(DPT * s, DPT)])


def _edge_scatter(h, src, dst, a_s, a_d, m16):
    mesh = plsc.VectorSubcoreMesh(core_axis_name="c", subcore_axis_name="s",
                                  num_cores=NC, num_subcores=NS)
    kern = pl.kernel(
        _edge_body,
        out_type=(
            jax.ShapeDtypeStruct((NC, NPAD, D), jnp.float32),
            jax.ShapeDtypeStruct((NC, NPAD), jnp.float32),
        ),
        mesh=mesh,
        compiler_params=pltpu.CompilerParams(needs_layout_passes=False),
        scratch_types=[
            pltpu.VMEM((16,), jnp.float32),
            pltpu.VMEM((3, BLK), jnp.int32),
            pltpu.VMEM((3, BLK), jnp.int32),
            pltpu.VMEM((3, BLK), jnp.float32),
            pltpu.VMEM((3, BLK), jnp.float32),
            pltpu.VMEM((3, BLK), jnp.float32),
            pltpu.VMEM((3, BLK, D), jnp.float32),
            pltpu.VMEM_SHARED((NPAD, D), jnp.float32),
            pltpu.VMEM_SHARED((NPAD,), jnp.float32),
            pltpu.SemaphoreType.DMA,
            pltpu.SemaphoreType.DMA,
            pltpu.SemaphoreType.DMA,
            pltpu.SemaphoreType.DMA,
        ],
    )
    return kern(h, src, dst, a_s, a_d, m16)


def _final_body(p_ref, d_ref, h_ref, as_ref, ad_ref, m_ref,
                bias_ref, gamma_ref, beta_ref, y_ref):
    mval = m_ref[0, 0]
    a = as_ref[...] + ad_ref[...]
    e = jnp.where(a > 0, a, _NEG_SLOPE * a)
    wl = jnp.exp(e - mval)
    raw = p_ref[0] + p_ref[1] + wl * h_ref[...]
    den = d_ref[0] + d_ref[1] + wl
    z = raw / (den + 1e-16) + bias_ref[...]
    mean = jnp.mean(z, axis=0, keepdims=True)
    var = jnp.mean((z - mean) ** 2, axis=0, keepdims=True)
    y = (z - mean) * lax.rsqrt(var + 1e-5) * gamma_ref[...] + beta_ref[...]
    y_ref[...] = jnp.maximum(y, 0.0)


def _finalize(partials, den2, h, a_s, a_d, m, bias, gamma, beta):
    return pl.pallas_call(
        _final_body,
        in_specs=[
            pl.BlockSpec((NC, N, D), lambda: (0, 0, 0)),
            pl.BlockSpec((NC, N, 1), lambda: (0, 0, 0)),
            pl.BlockSpec((N, D), lambda: (0, 0)),
            pl.BlockSpec((N, 1), lambda: (0, 0)),
            pl.BlockSpec((N, 1), lambda: (0, 0)),
            pl.BlockSpec(memory_space=pltpu.SMEM),
            pl.BlockSpec((1, D), lambda: (0, 0)),
            pl.BlockSpec((1, D), lambda: (0, 0)),
            pl.BlockSpec((1, D), lambda: (0, 0)),
        ],
        out_specs=pl.BlockSpec((N, D), lambda: (0, 0)),
        out_shape=jax.ShapeDtypeStruct((N, D), jnp.float32),
    )(partials, den2, h, a_s, a_d, m, bias, gamma, beta)


def kernel(x, edge_index, W, att_src, att_dst, bias, gamma, beta):
    h, a_s, a_d, m = _project(x, W,
                              att_src.reshape(D, 1), att_dst.reshape(D, 1))
    m16 = jnp.broadcast_to(m.reshape(()), (16,))
    src = edge_index[0]
    dst = edge_index[1]
    partials, denp = _edge_scatter(h, src, dst,
                                   a_s.reshape(N), a_d.reshape(N), m16)
    partials = partials[:, :N]
    den2 = denp[:, :N].reshape(NC, N, 1)
    return _finalize(partials, den2, h, a_s, a_d, m,
                     bias.reshape(1, D), gamma.reshape(1, D),
                     beta.reshape(1, D))


# speedup vs baseline: 46.5117x; 1.0249x over previous
import functools

import jax
import jax.numpy as jnp
from jax import lax
from jax.experimental import pallas as pl
from jax.experimental.pallas import tpu as pltpu
from jax.experimental.pallas import tpu_sc as plsc

N = 10000
D = 128
E = 320000

NC = 2
NS = 16
NW = NC * NS
EW = E // NW
BLK = 80
NBLK = EW // BLK
NPAD = 10240
NPT = NPAD // NS
DPT = NPAD // NS

_NEG_SLOPE = 0.2


def _proj_body(x_ref, w_ref, asrc_ref, adst_ref,
               h_ref, as_ref, ad_ref, m_ref, ms_ref, md_ref):
    i = pl.program_id(0)
    h = jnp.dot(x_ref[...], w_ref[...], preferred_element_type=jnp.float32)
    h_ref[...] = h
    a_s = jnp.dot(h, asrc_ref[...], preferred_element_type=jnp.float32)
    a_d = jnp.dot(h, adst_ref[...], preferred_element_type=jnp.float32)
    as_ref[...] = a_s
    ad_ref[...] = a_d
    bs = jnp.max(a_s)
    bd = jnp.max(a_d)

    @pl.when(i == 0)
    def _():
        ms_ref[0] = bs
        md_ref[0] = bd

    @pl.when(i > 0)
    def _():
        ms_ref[0] = jnp.maximum(ms_ref[0], bs)
        md_ref[0] = jnp.maximum(md_ref[0], bd)

    @pl.when(i == pl.num_programs(0) - 1)
    def _():
        t = ms_ref[0] + md_ref[0]
        m_ref[0, 0] = jnp.where(t > 0, t, _NEG_SLOPE * t)


def _project(x, w, asrc, adst):
    rows = 1000
    grid = (N // rows,)
    return pl.pallas_call(
        _proj_body,
        grid=grid,
        in_specs=[
            pl.BlockSpec((rows, D), lambda i: (i, 0)),
            pl.BlockSpec((D, D), lambda i: (0, 0)),
            pl.BlockSpec((D, 1), lambda i: (0, 0)),
            pl.BlockSpec((D, 1), lambda i: (0, 0)),
        ],
        out_specs=[
            pl.BlockSpec((rows, D), lambda i: (i, 0)),
            pl.BlockSpec((rows, 1), lambda i: (i, 0)),
            pl.BlockSpec((rows, 1), lambda i: (i, 0)),
            pl.BlockSpec(memory_space=pltpu.SMEM),
        ],
        out_shape=[
            jax.ShapeDtypeStruct((N, D), jnp.float32),
            jax.ShapeDtypeStruct((N, 1), jnp.float32),
            jax.ShapeDtypeStruct((N, 1), jnp.float32),
            jax.ShapeDtypeStruct((1, 1), jnp.float32),
        ],
        scratch_shapes=[
            pltpu.SMEM((1,), jnp.float32),
            pltpu.SMEM((1,), jnp.float32),
        ],
    )(x, w, asrc, adst)


def _edge_body(h_hbm, src_hbm, dst_hbm, as_hbm, ad_hbm, m_hbm,
               out_hbm, den_hbm,
               mv, sidx, didx, av_b, bv_b, wv, rows,
               acc_s, den_s, semi, sema, semg, sems):
    c = lax.axis_index("c")
    s = lax.axis_index("s")
    wid = s * NC + c
    z16f = jnp.zeros((16,), jnp.float32)

    pltpu.sync_copy(m_hbm, mv)
    mvec = mv[...]

    def zero_rows(i, carry):
        for dch in range(8):
            rows[0, i, pl.ds(dch * 16, 16)] = z16f
        return carry
    lax.fori_loop(0, BLK, zero_rows, 0)
    for q in range(BLK // 16):
        wv[0, pl.ds(16 * q, 16)] = z16f
    for off in range(0, NPT, BLK):
        pltpu.sync_copy(rows.at[0], acc_s.at[pl.ds(NPT * s + off, BLK)])
        pltpu.sync_copy(wv.at[0], den_s.at[pl.ds(DPT * s + off, BLK)])
    plsc.subcore_barrier()

    base0 = wid * EW

    def issue_idx(blk, p):
        base = base0 + blk * BLK
        pltpu.async_copy(src_hbm.at[pl.ds(base, BLK)], sidx.at[p], semi)
        pltpu.async_copy(dst_hbm.at[pl.ds(base, BLK)], didx.at[p], semi)

    def wait_idx(blk, p):
        base = base0 + blk * BLK
        pltpu.make_async_copy(src_hbm.at[pl.ds(base, BLK)], sidx.at[p],
                              semi).wait()
        pltpu.make_async_copy(dst_hbm.at[pl.ds(base, BLK)], didx.at[p],
                              semi).wait()

    def issue_gathers(p):
        pltpu.async_copy(h_hbm.at[sidx.at[p]], rows.at[p], semg)
        pltpu.async_copy(as_hbm.at[sidx.at[p]], av_b.at[p], sema)
        pltpu.async_copy(ad_hbm.at[didx.at[p]], bv_b.at[p], sema)

    def wait_agather(p):
        pltpu.make_async_copy(as_hbm.at[sidx.at[p]], av_b.at[p],
                              sema).wait()
        pltpu.make_async_copy(ad_hbm.at[didx.at[p]], bv_b.at[p],
                              sema).wait()

    def wait_rgather(p):
        pltpu.make_async_copy(h_hbm.at[sidx.at[p]], rows.at[p],
                              semg).wait()

    def wait_scat(p):
        pltpu.make_async_copy(rows.at[p], acc_s.at[didx.at[p]],
                              sems).wait()
        pltpu.make_async_copy(wv.at[p], den_s.at[didx.at[p]],
                              sems).wait()

    def do_block(blk, p, first=False, last=False, prefetch2=True):
        p1 = (p + 1) % 3
        p2 = (p + 2) % 3
        if not last:
            wait_idx(blk + 1, p1)
            issue_gathers(p1)

        wait_agather(p)
        for q in range(BLK // 16):
            e = av_b[p, pl.ds(16 * q, 16)] + bv_b[p, pl.ds(16 * q, 16)]
            e = jnp.where(e > 0, e, _NEG_SLOPE * e)
            wv[p, pl.ds(16 * q, 16)] = jnp.exp(e - mvec)

        wait_rgather(p)
        pvec = jnp.full((16,), p, jnp.int32)

        def mul_body(ei, carry2):
            for u in range(4):
                eu = 4 * ei + u
                wb = plsc.load_gather(wv, [pvec, jnp.full((16,), eu,
                                                          jnp.int32)])
                for dch in range(8):
                    sl = pl.ds(dch * 16, 16)
                    rows[p, eu, sl] = rows[p, eu, sl] * wb
            return carry2
        lax.fori_loop(0, BLK // 4, mul_body, 0)

        if not first:
            wait_scat(p2)
        if prefetch2 and not last:
            issue_idx(blk + 2, p2)

        pltpu.async_copy(rows.at[p], acc_s.at[didx.at[p]], sems, add=True)
        pltpu.async_copy(wv.at[p], den_s.at[didx.at[p]], sems, add=True)

    issue_idx(0, 0)
    wait_idx(0, 0)
    issue_gathers(0)
    issue_idx(1, 1)
    do_block(0, 0, first=True)

    def loop_body(g, carry):
        do_block(3 * g + 1, 1)
        do_block(3 * g + 2, 2)
        do_block(3 * g + 3, 0)
        return carry

    lax.fori_loop(0, (NBLK - 5) // 3, loop_body, 0)
    do_block(NBLK - 4, 1)
    do_block(NBLK - 3, 2)
    do_block(NBLK - 2, 0, prefetch2=False)
    do_block(NBLK - 1, 1, last=True)
    wait_scat(1)

    plsc.subcore_barrier()
    pltpu.sync_copy(acc_s.at[pl.ds(NPT * s, NPT)],
                    out_hbm.at[c, pl.ds(NPT * s, NPT)])
    pltpu.sync_copy(den_s.at[pl.ds(DPT * s, DPT)],
                    den_hbm.at[c, pl.ds(DPT * s, DPT)])


def _edge_scatter(h, src, dst, a_s, a_d, m16):
    mesh = plsc.VectorSubcoreMesh(core_axis_name="c", subcore_axis_name="s",
                                  num_cores=NC, num_subcores=NS)
    kern = pl.kernel(
        _edge_body,
        out_type=(
            jax.ShapeDtypeStruct((NC, NPAD, D), jnp.float32),
            jax.ShapeDtypeStruct((NC, NPAD), jnp.float32),
        ),
        mesh=mesh,
        compiler_params=pltpu.CompilerParams(needs_layout_passes=False),
        scratch_types=[
            pltpu.VMEM((16,), jnp.float32),
            pltpu.VMEM((3, BLK), jnp.int32),
            pltpu.VMEM((3, BLK), jnp.int32),
            pltpu.VMEM((3, BLK), jnp.float32),
            pltpu.VMEM((3, BLK), jnp.float32),
            pltpu.VMEM((3, BLK), jnp.float32),
            pltpu.VMEM((3, BLK, D), jnp.float32),
            pltpu.VMEM_SHARED((NPAD, D), jnp.float32),
            pltpu.VMEM_SHARED((NPAD,), jnp.float32),
            pltpu.SemaphoreType.DMA,
            pltpu.SemaphoreType.DMA,
            pltpu.SemaphoreType.DMA,
            pltpu.SemaphoreType.DMA,
        ],
    )
    return kern(h, src, dst, a_s, a_d, m16)


def _final_body(p_ref, d_ref, h_ref, as_ref, ad_ref, m_ref,
                bias_ref, gamma_ref, beta_ref, y_ref):
    mval = m_ref[0, 0]
    a = as_ref[...] + ad_ref[...]
    e = jnp.where(a > 0, a, _NEG_SLOPE * a)
    wl = jnp.exp(e - mval)
    raw = p_ref[0, :N] + p_ref[1, :N] + wl * h_ref[...]
    den = d_ref[0, :N] + d_ref[1, :N] + wl
    z = raw / (den + 1e-16) + bias_ref[...]
    mean = jnp.mean(z, axis=0, keepdims=True)
    var = jnp.mean((z - mean) ** 2, axis=0, keepdims=True)
    y = (z - mean) * lax.rsqrt(var + 1e-5) * gamma_ref[...] + beta_ref[...]
    y_ref[...] = jnp.maximum(y, 0.0)


def _finalize(partials, den2, h, a_s, a_d, m, bias, gamma, beta):
    return pl.pallas_call(
        _final_body,
        in_specs=[
            pl.BlockSpec((NC, NPAD, D), lambda: (0, 0, 0)),
            pl.BlockSpec((NC, NPAD, 1), lambda: (0, 0, 0)),
            pl.BlockSpec((N, D), lambda: (0, 0)),
            pl.BlockSpec((N, 1), lambda: (0, 0)),
            pl.BlockSpec((N, 1), lambda: (0, 0)),
            pl.BlockSpec(memory_space=pltpu.SMEM),
            pl.BlockSpec((1, D), lambda: (0, 0)),
            pl.BlockSpec((1, D), lambda: (0, 0)),
            pl.BlockSpec((1, D), lambda: (0, 0)),
        ],
        out_specs=pl.BlockSpec((N, D), lambda: (0, 0)),
        out_shape=jax.ShapeDtypeStruct((N, D), jnp.float32),
    )(partials, den2, h, a_s, a_d, m, bias, gamma, beta)


def kernel(x, edge_index, W, att_src, att_dst, bias, gamma, beta):
    h, a_s, a_d, m = _project(x, W,
                              att_src.reshape(D, 1), att_dst.reshape(D, 1))
    m16 = jnp.broadcast_to(m.reshape(()), (16,))
    src = edge_index[0]
    dst = edge_index[1]
    partials, denp = _edge_scatter(h, src, dst,
                                   a_s.reshape(N), a_d.reshape(N), m16)
    den2 = denp.reshape(NC, NPAD, 1)
    return _finalize(partials, den2, h, a_s, a_d, m,
                     bias.reshape(1, D), gamma.reshape(1, D),
                     beta.reshape(1, D))
